# baseline (speedup 1.0000x reference)
import jax
import jax.numpy as jnp
from jax.experimental import pallas as pl
from jax.experimental.pallas import tpu as pltpu

_ZDIM = 128
_BM = 1024


def _adj_kernel(z1_ref, z2_ref, rk_ref, adj_ref, z1_out_ref, z2_out_ref,
                rk_out_ref):
    b = pl.program_id(0)
    i = pl.program_id(1)

    half_lgt = jax.lax.dot_general(
        z1_ref[0] * 0.5,
        z2_ref[0],
        (((1,), (1,)), ((), ())),
        preferred_element_type=jnp.float32,
    )
    adj_ref[0] = 0.5 * jnp.tanh(half_lgt) + 0.5

    z1_out_ref[...] = z1_ref[...]

    @pl.when(i == 0)
    def _():
        z2_out_ref[...] = z2_ref[...]

    @pl.when((b == 0) & (i == 0))
    def _():
        rk_out_ref[...] = jax.nn.sigmoid(rk_ref[...])


def kernel(z1, z2, rk_lgt):
    b_dim, n, d = z1.shape
    grid = (b_dim, n // _BM)
    adj, z1_out, z2_out, rk_sq = pl.pallas_call(
        _adj_kernel,
        grid=grid,
        in_specs=[
            pl.BlockSpec((1, _BM, d), lambda b, i: (b, i, 0)),
            pl.BlockSpec((1, n, d), lambda b, i: (b, 0, 0)),
            pl.BlockSpec((1, _ZDIM), lambda b, i: (0, 0)),
        ],
        out_specs=[
            pl.BlockSpec((1, _BM, n), lambda b, i: (b, i, 0)),
            pl.BlockSpec((1, _BM, d), lambda b, i: (b, i, 0)),
            pl.BlockSpec((1, n, d), lambda b, i: (b, 0, 0)),
            pl.BlockSpec((1, _ZDIM), lambda b, i: (0, 0)),
        ],
        out_shape=[
            jax.ShapeDtypeStruct((b_dim, n, n), jnp.float32),
            jax.ShapeDtypeStruct((b_dim, n, d), jnp.float32),
            jax.ShapeDtypeStruct((b_dim, n, d), jnp.float32),
            jax.ShapeDtypeStruct((1, _ZDIM), jnp.float32),
        ],
        compiler_params=pltpu.CompilerParams(
            dimension_semantics=("parallel", "parallel"),
        ),
    )(z1, z2, rk_lgt)
    return (adj, z1_out, z2_out, rk_sq)


# speedup vs baseline: 1.0121x; 1.0121x over previous
import jax
import jax.numpy as jnp
from jax.experimental import pallas as pl
from jax.experimental.pallas import tpu as pltpu

_ZDIM = 128
_BM = 512


def _adj_kernel(z1_ref, z2_ref, rk_ref, adj_ref, z1_out_ref, z2_out_ref,
                rk_out_ref):
    b = pl.program_id(0)
    i = pl.program_id(1)

    half_lgt = jax.lax.dot_general(
        z1_ref[0] * 0.5,
        z2_ref[0],
        (((1,), (1,)), ((), ())),
        preferred_element_type=jnp.float32,
    )
    adj_ref[0] = 0.5 * jnp.tanh(half_lgt) + 0.5

    z1_out_ref[...] = z1_ref[...]

    @pl.when(i == 0)
    def _():
        z2_out_ref[...] = z2_ref[...]

    @pl.when((b == 0) & (i == 0))
    def _():
        rk_out_ref[...] = jax.nn.sigmoid(rk_ref[...])


def kernel(z1, z2, rk_lgt):
    b_dim, n, d = z1.shape
    grid = (b_dim, n // _BM)
    adj, z1_out, z2_out, rk_sq = pl.pallas_call(
        _adj_kernel,
        grid=grid,
        in_specs=[
            pl.BlockSpec((1, _BM, d), lambda b, i: (b, i, 0)),
            pl.BlockSpec((1, n, d), lambda b, i: (b, 0, 0)),
            pl.BlockSpec((1, _ZDIM), lambda b, i: (0, 0)),
        ],
        out_specs=[
            pl.BlockSpec((1, _BM, n), lambda b, i: (b, i, 0)),
            pl.BlockSpec((1, _BM, d), lambda b, i: (b, i, 0)),
            pl.BlockSpec((1, n, d), lambda b, i: (b, 0, 0)),
            pl.BlockSpec((1, _ZDIM), lambda b, i: (0, 0)),
        ],
        out_shape=[
            jax.ShapeDtypeStruct((b_dim, n, n), jnp.float32),
            jax.ShapeDtypeStruct((b_dim, n, d), jnp.float32),
            jax.ShapeDtypeStruct((b_dim, n, d), jnp.float32),
            jax.ShapeDtypeStruct((1, _ZDIM), jnp.float32),
        ],
        compiler_params=pltpu.CompilerParams(
            dimension_semantics=("parallel", "parallel"),
        ),
    )(z1, z2, rk_lgt)
    return (adj, z1_out, z2_out, rk_sq)
